# baseline (speedup 1.0000x reference)
import functools

import jax
import jax.numpy as jnp
from jax.experimental import pallas as pl
from jax.experimental.pallas import tpu as pltpu


def _fused_kernel(x_ref, perm_ref, wtoe_ref, wtail_ref, bconv_ref,
                  whidT_ref, bhid_ref, wneuT_ref, bneu_ref, out_ref,
                  xc_ref, x2_ref,
                  *, cin, m, p, s, nfull, tail_start):
    B = x_ref.shape[0]
    R = x_ref.shape[1]
    dn = (((1,), (1,)), ((), ()))

    xc_ref[...] = x_ref[...].astype(jnp.bfloat16)
    xc = xc_ref[...]
    for j in range(R // 128):
        x2_ref[j * 128:(j + 1) * 128, :] = jax.lax.dot_general(
            perm_ref[j * 128:(j + 1) * 128, :], xc, dn,
            preferred_element_type=jnp.float32).astype(jnp.bfloat16)

    wtoe = wtoe_ref[...]
    stride = p * cin
    feat = jnp.full((p * m, B), -1e30, jnp.float32)
    for c in range(nfull):
        xs = x2_ref[pl.ds(c * stride, s), :]
        feat = jnp.maximum(feat, jnp.dot(wtoe, xs,
                                         preferred_element_type=jnp.float32))
    if tail_start is not None:
        xs = x2_ref[pl.ds(tail_start, s), :]
        feat = jnp.maximum(feat, jnp.dot(wtail_ref[...], xs,
                                         preferred_element_type=jnp.float32))

    acc = feat[0:m, :]
    for q in range(1, p):
        acc = jnp.maximum(acc, feat[q * m:(q + 1) * m, :])
    acc = jnp.maximum(acc + bconv_ref[...], 0.0)

    h = jnp.dot(whidT_ref[...], acc,
                preferred_element_type=jnp.float32) + bhid_ref[...]
    h = jnp.maximum(h, 0.0)
    logit = jnp.dot(wneuT_ref[...], h,
                    preferred_element_type=jnp.float32) + bneu_ref[...]
    out_ref[...] = jax.nn.sigmoid(logit)


def _toeplitz(wflat, m, p, s, cin, shift):
    wt = jnp.zeros((p * m, s), jnp.float32)
    for q in range(p):
        wt = jax.lax.dynamic_update_slice(wt, wflat, (q * m, shift + q * cin))
    return wt.astype(jnp.bfloat16)


def kernel(x, wconv, bconv, whid, bhid, wneu, bneu, *, block_b=512):
    N, cin, L = x.shape
    M, _, K = wconv.shape
    H = whid.shape[1]
    lout = L - K + 1
    P = 128 // M
    S = ((P - 1) * cin + cin * K + 127) // 128 * 128
    R = L * cin
    assert R % 128 == 0
    nfull = lout // P
    assert nfull >= 1
    ntail = lout - nfull * P

    if ntail:
        l0 = lout - P
        tail_start = l0 * cin // 16 * 16
        shift = l0 * cin - tail_start
        assert shift + (P - 1) * cin + K * cin <= S
        assert tail_start + S <= R
    else:
        tail_start, shift = None, 0

    npad = max(block_b, (N + block_b - 1) // block_b * block_b)
    xr = x.reshape(N, R)
    if npad != N:
        xr = jnp.pad(xr, ((0, npad - N), (0, 0)))

    rows = jnp.arange(R)
    perm = jnp.zeros((R, R), jnp.float32).at[
        rows, (rows % cin) * L + rows // cin].set(1.0).astype(jnp.bfloat16)

    wflat = jnp.transpose(wconv.astype(jnp.float32), (0, 2, 1)).reshape(M, K * cin)
    wtoe = _toeplitz(wflat, M, P, S, cin, 0)
    wtail = _toeplitz(wflat, M, P, S, cin, shift) if ntail else wtoe

    bconv2 = bconv.reshape(M, 1).astype(jnp.float32)
    whidT = whid.T.astype(jnp.float32)
    bhid2 = bhid.reshape(H, 1).astype(jnp.float32)
    wneuT = wneu.T.astype(jnp.float32)
    bneu2 = bneu.reshape(1, 1).astype(jnp.float32)

    kfn = functools.partial(_fused_kernel, cin=cin, m=M, p=P, s=S,
                            nfull=nfull, tail_start=tail_start)
    out = pl.pallas_call(
        kfn,
        out_shape=jax.ShapeDtypeStruct((1, npad), jnp.float32),
        grid_spec=pltpu.PrefetchScalarGridSpec(
            num_scalar_prefetch=0,
            grid=(npad // block_b,),
            in_specs=[
                pl.BlockSpec((block_b, R), lambda n: (n, 0)),
                pl.BlockSpec((R, R), lambda n: (0, 0)),
                pl.BlockSpec((P * M, S), lambda n: (0, 0)),
                pl.BlockSpec((P * M, S), lambda n: (0, 0)),
                pl.BlockSpec((M, 1), lambda n: (0, 0)),
                pl.BlockSpec((H, M), lambda n: (0, 0)),
                pl.BlockSpec((H, 1), lambda n: (0, 0)),
                pl.BlockSpec((1, H), lambda n: (0, 0)),
                pl.BlockSpec((1, 1), lambda n: (0, 0)),
            ],
            out_specs=pl.BlockSpec((1, block_b), lambda n: (0, n)),
            scratch_shapes=[
                pltpu.VMEM((block_b, R), jnp.bfloat16),
                pltpu.VMEM((R, block_b), jnp.bfloat16),
            ],
        ),
        compiler_params=pltpu.CompilerParams(
            dimension_semantics=("parallel",)),
    )(xr, perm, wtoe, wtail, bconv2, whidT, bhid2, wneuT, bneu2)
    return out[0, :N].reshape(N, 1)


# speedup vs baseline: 1.6146x; 1.6146x over previous
import functools

import jax
import jax.numpy as jnp
from jax.experimental import pallas as pl
from jax.experimental.pallas import tpu as pltpu


def _fused_kernel(x_hbm, perm_ref, wtoe_ref, wtail_ref, bconv_ref,
                  whidT_ref, bhid_ref, wneuT_ref, bneu_ref, out_ref,
                  bufs, sems, x2_ref,
                  *, cin, m, p, s, nfull, tail_start, block_b):
    j = pl.program_id(1)
    g2 = pl.num_programs(1)
    base = (pl.program_id(0) * g2 + j) * block_b
    R = x2_ref.shape[0]
    B = block_b

    @pl.when(j == 0)
    def _():
        for c in range(cin):
            pltpu.make_async_copy(x_hbm.at[pl.ds(base, B), c],
                                  bufs.at[0, c], sems.at[0, c]).start()

    @pl.when(j + 1 < g2)
    def _():
        nbase = base + B
        nslot = (j + 1) % 2
        for c in range(cin):
            pltpu.make_async_copy(x_hbm.at[pl.ds(nbase, B), c],
                                  bufs.at[nslot, c], sems.at[nslot, c]).start()

    slot = j % 2
    for c in range(cin):
        pltpu.make_async_copy(bufs.at[slot, c], bufs.at[slot, c],
                              sems.at[slot, c]).wait()

    dn = (((1,), (1,)), ((), ()))
    xcb = [bufs[slot, c].astype(jnp.bfloat16) for c in range(cin)]
    for grp in range(R // 128):
        y = None
        for c in range(cin):
            pj = perm_ref[(grp * cin + c) * 128:(grp * cin + c + 1) * 128, :]
            d = jax.lax.dot_general(pj, xcb[c], dn,
                                    preferred_element_type=jnp.float32)
            y = d if y is None else y + d
        x2_ref[grp * 128:(grp + 1) * 128, :] = y.astype(jnp.bfloat16)

    wtoe = wtoe_ref[...]
    stride = p * cin
    feat = jnp.full((p * m, B), -1e30, jnp.float32)
    for c in range(nfull):
        xs = x2_ref[pl.ds(c * stride, s), :]
        feat = jnp.maximum(feat, jnp.dot(wtoe, xs,
                                         preferred_element_type=jnp.float32))
    if tail_start is not None:
        xs = x2_ref[pl.ds(tail_start, s), :]
        feat = jnp.maximum(feat, jnp.dot(wtail_ref[...], xs,
                                         preferred_element_type=jnp.float32))

    acc = feat[0:m, :]
    for q in range(1, p):
        acc = jnp.maximum(acc, feat[q * m:(q + 1) * m, :])
    acc = jnp.maximum(acc + bconv_ref[...], 0.0)

    h = jnp.dot(whidT_ref[...], acc,
                preferred_element_type=jnp.float32) + bhid_ref[...]
    h = jnp.maximum(h, 0.0)
    logit = jnp.dot(wneuT_ref[...], h,
                    preferred_element_type=jnp.float32) + bneu_ref[...]
    out_ref[...] = jax.nn.sigmoid(logit)


def _toeplitz(wflat, m, p, s, cin, shift):
    kc = wflat.shape[1]
    colidx = jnp.arange(s)[None, :] - jnp.arange(p)[:, None] * cin - shift
    valid = (colidx >= 0) & (colidx < kc)
    g = wflat[:, jnp.clip(colidx, 0, kc - 1)]
    wt = jnp.where(valid[None], g, 0.0)
    return wt.transpose(1, 0, 2).reshape(p * m, s).astype(jnp.bfloat16)


def kernel(x, wconv, bconv, whid, bhid, wneu, bneu, *, block_b=512):
    N, cin, L = x.shape
    M, _, K = wconv.shape
    H = whid.shape[1]
    lout = L - K + 1
    P = 128 // M
    S = ((P - 1) * cin + cin * K + 127) // 128 * 128
    R = L * cin
    assert R % 128 == 0 and 128 % cin == 0
    nfull = lout // P
    assert nfull >= 1
    ntail = lout - nfull * P

    if ntail:
        l0 = lout - P
        tail_start = l0 * cin // 16 * 16
        shift = l0 * cin - tail_start
        assert shift + (P - 1) * cin + K * cin <= S
        assert tail_start + S <= R
    else:
        tail_start, shift = None, 0

    npad = max(2 * block_b, (N + 2 * block_b - 1) // (2 * block_b) * (2 * block_b))
    if npad != N:
        x = jnp.pad(x, ((0, npad - N), (0, 0), (0, 0)))

    ngrp = R // 128
    gidx = jnp.arange(ngrp * cin)[:, None]
    jj, cc = gidx // cin, gidx % cin
    r = jnp.arange(128)[None, :]
    tgt = jnp.where((r % cin) == cc, r // cin + (128 // cin) * jj, -1)
    perm = jax.nn.one_hot(tgt, L, dtype=jnp.float32).astype(
        jnp.bfloat16).reshape(ngrp * cin * 128, L)

    wflat = jnp.transpose(wconv.astype(jnp.float32), (0, 2, 1)).reshape(M, K * cin)
    wtoe = _toeplitz(wflat, M, P, S, cin, 0)
    wtail = _toeplitz(wflat, M, P, S, cin, shift) if ntail else wtoe

    bconv2 = bconv.reshape(M, 1).astype(jnp.float32)
    whidT = whid.T.astype(jnp.float32)
    bhid2 = bhid.reshape(H, 1).astype(jnp.float32)
    wneuT = wneu.T.astype(jnp.float32)
    bneu2 = bneu.reshape(1, 1).astype(jnp.float32)

    g2 = npad // (2 * block_b)
    kfn = functools.partial(_fused_kernel, cin=cin, m=M, p=P, s=S,
                            nfull=nfull, tail_start=tail_start,
                            block_b=block_b)
    out = pl.pallas_call(
        kfn,
        out_shape=jax.ShapeDtypeStruct((1, npad), jnp.float32),
        grid_spec=pltpu.PrefetchScalarGridSpec(
            num_scalar_prefetch=0,
            grid=(2, g2),
            in_specs=[
                pl.BlockSpec(memory_space=pl.ANY),
                pl.BlockSpec((ngrp * cin * 128, L), lambda i, j: (0, 0)),
                pl.BlockSpec((P * M, S), lambda i, j: (0, 0)),
                pl.BlockSpec((P * M, S), lambda i, j: (0, 0)),
                pl.BlockSpec((M, 1), lambda i, j: (0, 0)),
                pl.BlockSpec((H, M), lambda i, j: (0, 0)),
                pl.BlockSpec((H, 1), lambda i, j: (0, 0)),
                pl.BlockSpec((1, H), lambda i, j: (0, 0)),
                pl.BlockSpec((1, 1), lambda i, j: (0, 0)),
            ],
            out_specs=pl.BlockSpec((1, block_b),
                                   lambda i, j, g2=g2: (0, i * g2 + j)),
            scratch_shapes=[
                pltpu.VMEM((2, cin, block_b, L), jnp.float32),
                pltpu.SemaphoreType.DMA((2, cin)),
                pltpu.VMEM((R, block_b), jnp.bfloat16),
            ],
        ),
        compiler_params=pltpu.CompilerParams(
            dimension_semantics=("parallel", "arbitrary")),
    )(x, perm, wtoe, wtail, bconv2, whidT, bhid2, wneuT, bneu2)
    return out[0, :N].reshape(N, 1)


# speedup vs baseline: 1.8068x; 1.1191x over previous
import functools

import jax
import jax.numpy as jnp
from jax.experimental import pallas as pl
from jax.experimental.pallas import tpu as pltpu


def _fused_kernel(x_hbm, perm_ref, wtoe_ref, wtail_ref, bconv_ref,
                  whidT_ref, bhid_ref, wneuT_ref, bneu_ref, out_ref,
                  bufs, sems, x2_ref,
                  *, cin, m, p, s, nfull, tail_start, block_b):
    j = pl.program_id(1)
    g2 = pl.num_programs(1)
    base = (pl.program_id(0) * g2 + j) * block_b
    R = x2_ref.shape[0]
    B = block_b

    @pl.when(j == 0)
    def _():
        for c in range(cin):
            pltpu.make_async_copy(x_hbm.at[pl.ds(base, B), c],
                                  bufs.at[0, c], sems.at[0, c]).start()

    @pl.when(j + 1 < g2)
    def _():
        nbase = base + B
        nslot = (j + 1) % 2
        for c in range(cin):
            pltpu.make_async_copy(x_hbm.at[pl.ds(nbase, B), c],
                                  bufs.at[nslot, c], sems.at[nslot, c]).start()

    slot = j % 2
    for c in range(cin):
        pltpu.make_async_copy(bufs.at[slot, c], bufs.at[slot, c],
                              sems.at[slot, c]).wait()

    dn = (((1,), (1,)), ((), ()))
    xcb = [bufs[slot, c].astype(jnp.bfloat16) for c in range(cin)]
    for grp in range(R // 128):
        y = None
        for c in range(cin):
            pj = perm_ref[(grp * cin + c) * 128:(grp * cin + c + 1) * 128, :]
            d = jax.lax.dot_general(pj, xcb[c], dn,
                                    preferred_element_type=jnp.float32)
            y = d if y is None else y + d
        x2_ref[grp * 128:(grp + 1) * 128, :] = y.astype(jnp.bfloat16)

    wtoe = wtoe_ref[...]
    stride = p * cin
    feat = jnp.full((p * m, B), -1e30, jnp.float32)
    for c in range(nfull):
        xs = x2_ref[pl.ds(c * stride, s), :]
        feat = jnp.maximum(feat, jnp.dot(wtoe, xs,
                                         preferred_element_type=jnp.float32))
    if tail_start is not None:
        xs = x2_ref[pl.ds(tail_start, s), :]
        feat = jnp.maximum(feat, jnp.dot(wtail_ref[...], xs,
                                         preferred_element_type=jnp.float32))

    acc = feat[0:m, :]
    for q in range(1, p):
        acc = jnp.maximum(acc, feat[q * m:(q + 1) * m, :])
    acc = jnp.maximum(acc + bconv_ref[...], 0.0)

    h = jnp.dot(whidT_ref[...], acc,
                preferred_element_type=jnp.float32) + bhid_ref[...]
    h = jnp.maximum(h, 0.0)
    logit = jnp.dot(wneuT_ref[...], h,
                    preferred_element_type=jnp.float32) + bneu_ref[...]
    out_ref[...] = jax.nn.sigmoid(logit)


def _toeplitz(wflat, m, p, s, cin, shift):
    kc = wflat.shape[1]
    colidx = jnp.arange(s)[None, :] - jnp.arange(p)[:, None] * cin - shift
    valid = (colidx >= 0) & (colidx < kc)
    g = wflat[:, jnp.clip(colidx, 0, kc - 1)]
    wt = jnp.where(valid[None], g, 0.0)
    return wt.transpose(1, 0, 2).reshape(p * m, s).astype(jnp.bfloat16)


def kernel(x, wconv, bconv, whid, bhid, wneu, bneu, *, block_b=1024):
    N, cin, L = x.shape
    M, _, K = wconv.shape
    H = whid.shape[1]
    lout = L - K + 1
    P = 128 // M
    S = ((P - 1) * cin + cin * K + 127) // 128 * 128
    R = L * cin
    assert R % 128 == 0 and 128 % cin == 0
    nfull = lout // P
    assert nfull >= 1
    ntail = lout - nfull * P

    if ntail:
        l0 = lout - P
        tail_start = l0 * cin // 16 * 16
        shift = l0 * cin - tail_start
        assert shift + (P - 1) * cin + K * cin <= S
        assert tail_start + S <= R
    else:
        tail_start, shift = None, 0

    npad = max(2 * block_b, (N + 2 * block_b - 1) // (2 * block_b) * (2 * block_b))
    if npad != N:
        x = jnp.pad(x, ((0, npad - N), (0, 0), (0, 0)))

    ngrp = R // 128
    gidx = jnp.arange(ngrp * cin)[:, None]
    jj, cc = gidx // cin, gidx % cin
    r = jnp.arange(128)[None, :]
    tgt = jnp.where((r % cin) == cc, r // cin + (128 // cin) * jj, -1)
    perm = jax.nn.one_hot(tgt, L, dtype=jnp.float32).astype(
        jnp.bfloat16).reshape(ngrp * cin * 128, L)

    wflat = jnp.transpose(wconv.astype(jnp.float32), (0, 2, 1)).reshape(M, K * cin)
    wtoe = _toeplitz(wflat, M, P, S, cin, 0)
    wtail = _toeplitz(wflat, M, P, S, cin, shift) if ntail else wtoe

    bconv2 = bconv.reshape(M, 1).astype(jnp.float32)
    whidT = whid.T.astype(jnp.float32)
    bhid2 = bhid.reshape(H, 1).astype(jnp.float32)
    wneuT = wneu.T.astype(jnp.float32)
    bneu2 = bneu.reshape(1, 1).astype(jnp.float32)

    g2 = npad // (2 * block_b)
    kfn = functools.partial(_fused_kernel, cin=cin, m=M, p=P, s=S,
                            nfull=nfull, tail_start=tail_start,
                            block_b=block_b)
    out = pl.pallas_call(
        kfn,
        out_shape=jax.ShapeDtypeStruct((1, npad), jnp.float32),
        grid_spec=pltpu.PrefetchScalarGridSpec(
            num_scalar_prefetch=0,
            grid=(2, g2),
            in_specs=[
                pl.BlockSpec(memory_space=pl.ANY),
                pl.BlockSpec((ngrp * cin * 128, L), lambda i, j: (0, 0)),
                pl.BlockSpec((P * M, S), lambda i, j: (0, 0)),
                pl.BlockSpec((P * M, S), lambda i, j: (0, 0)),
                pl.BlockSpec((M, 1), lambda i, j: (0, 0)),
                pl.BlockSpec((H, M), lambda i, j: (0, 0)),
                pl.BlockSpec((H, 1), lambda i, j: (0, 0)),
                pl.BlockSpec((1, H), lambda i, j: (0, 0)),
                pl.BlockSpec((1, 1), lambda i, j: (0, 0)),
            ],
            out_specs=pl.BlockSpec((1, block_b),
                                   lambda i, j, g2=g2: (0, i * g2 + j)),
            scratch_shapes=[
                pltpu.VMEM((2, cin, block_b, L), jnp.float32),
                pltpu.SemaphoreType.DMA((2, cin)),
                pltpu.VMEM((R, block_b), jnp.bfloat16),
            ],
        ),
        compiler_params=pltpu.CompilerParams(
            dimension_semantics=("parallel", "arbitrary")),
    )(x, perm, wtoe, wtail, bconv2, whidT, bhid2, wneuT, bneu2)
    return out[0, :N].reshape(N, 1)


# speedup vs baseline: 1.9036x; 1.0535x over previous
import functools

import jax
import jax.numpy as jnp
from jax.experimental import pallas as pl
from jax.experimental.pallas import tpu as pltpu


def _fused_kernel(x_hbm, perm_ref, wtoe_ref, wtail_ref, bconv_ref,
                  whidT_ref, bhid_ref, wneuT_ref, bneu_ref, out_ref,
                  bufs, sems, x2_ref,
                  *, cin, m, p, s, nfull, tail_start, block_b):
    j = pl.program_id(1)
    g2 = pl.num_programs(1)
    base = (pl.program_id(0) * g2 + j) * block_b
    R = x2_ref.shape[0]
    B = block_b

    @pl.when(j == 0)
    def _():
        for c in range(cin):
            pltpu.make_async_copy(x_hbm.at[pl.ds(base, B), c],
                                  bufs.at[0, c], sems.at[0, c]).start()

    @pl.when(j + 1 < g2)
    def _():
        nbase = base + B
        nslot = (j + 1) % 2
        for c in range(cin):
            pltpu.make_async_copy(x_hbm.at[pl.ds(nbase, B), c],
                                  bufs.at[nslot, c], sems.at[nslot, c]).start()

    slot = j % 2
    for c in range(cin):
        pltpu.make_async_copy(bufs.at[slot, c], bufs.at[slot, c],
                              sems.at[slot, c]).wait()

    dn = (((1,), (1,)), ((), ()))
    xcb = [bufs[slot, c].astype(jnp.bfloat16) for c in range(cin)]
    for grp in range(R // 128):
        y = None
        for c in range(cin):
            pj = perm_ref[(grp * cin + c) * 128:(grp * cin + c + 1) * 128, :]
            d = jax.lax.dot_general(pj, xcb[c], dn,
                                    preferred_element_type=jnp.float32)
            y = d if y is None else y + d
        x2_ref[grp * 128:(grp + 1) * 128, :] = y.astype(jnp.bfloat16)

    wtoe = wtoe_ref[...]
    stride = p * cin
    feat = jnp.full((p * m, B), -1e30, jnp.float32)
    for c in range(nfull):
        xs = x2_ref[pl.ds(c * stride, s), :]
        feat = jnp.maximum(feat, jnp.dot(wtoe, xs,
                                         preferred_element_type=jnp.float32))
    if tail_start is not None:
        xs = x2_ref[pl.ds(tail_start, s), :]
        feat = jnp.maximum(feat, jnp.dot(wtail_ref[...], xs,
                                         preferred_element_type=jnp.float32))

    acc = feat[0:m, :]
    for q in range(1, p):
        acc = jnp.maximum(acc, feat[q * m:(q + 1) * m, :])
    acc = jnp.maximum(acc + bconv_ref[...], 0.0)

    h = jnp.dot(whidT_ref[...], acc,
                preferred_element_type=jnp.float32) + bhid_ref[...]
    h = jnp.maximum(h, 0.0)
    logit = jnp.dot(wneuT_ref[...], h,
                    preferred_element_type=jnp.float32) + bneu_ref[...]
    out_ref[...] = jax.nn.sigmoid(logit)


def _toeplitz(wflat, m, p, s, cin, shift):
    kc = wflat.shape[1]
    colidx = jnp.arange(s)[None, :] - jnp.arange(p)[:, None] * cin - shift
    valid = (colidx >= 0) & (colidx < kc)
    g = wflat[:, jnp.clip(colidx, 0, kc - 1)]
    wt = jnp.where(valid[None], g, 0.0)
    return wt.transpose(1, 0, 2).reshape(p * m, s).astype(jnp.bfloat16)


def kernel(x, wconv, bconv, whid, bhid, wneu, bneu, *, block_b=2048):
    N, cin, L = x.shape
    M, _, K = wconv.shape
    H = whid.shape[1]
    lout = L - K + 1
    P = 128 // M
    S = ((P - 1) * cin + cin * K + 127) // 128 * 128
    R = L * cin
    assert R % 128 == 0 and 128 % cin == 0
    nfull = lout // P
    assert nfull >= 1
    ntail = lout - nfull * P

    if ntail:
        l0 = lout - P
        tail_start = l0 * cin // 16 * 16
        shift = l0 * cin - tail_start
        assert shift + (P - 1) * cin + K * cin <= S
        assert tail_start + S <= R
    else:
        tail_start, shift = None, 0

    npad = max(2 * block_b, (N + 2 * block_b - 1) // (2 * block_b) * (2 * block_b))
    if npad != N:
        x = jnp.pad(x, ((0, npad - N), (0, 0), (0, 0)))

    ngrp = R // 128
    gidx = jnp.arange(ngrp * cin)[:, None]
    jj, cc = gidx // cin, gidx % cin
    r = jnp.arange(128)[None, :]
    tgt = jnp.where((r % cin) == cc, r // cin + (128 // cin) * jj, -1)
    perm = jax.nn.one_hot(tgt, L, dtype=jnp.float32).astype(
        jnp.bfloat16).reshape(ngrp * cin * 128, L)

    wflat = jnp.transpose(wconv.astype(jnp.float32), (0, 2, 1)).reshape(M, K * cin)
    wtoe = _toeplitz(wflat, M, P, S, cin, 0)
    wtail = _toeplitz(wflat, M, P, S, cin, shift) if ntail else wtoe

    bconv2 = bconv.reshape(M, 1).astype(jnp.float32)
    whidT = whid.T.astype(jnp.float32)
    bhid2 = bhid.reshape(H, 1).astype(jnp.float32)
    wneuT = wneu.T.astype(jnp.float32)
    bneu2 = bneu.reshape(1, 1).astype(jnp.float32)

    g2 = npad // (2 * block_b)
    kfn = functools.partial(_fused_kernel, cin=cin, m=M, p=P, s=S,
                            nfull=nfull, tail_start=tail_start,
                            block_b=block_b)
    out = pl.pallas_call(
        kfn,
        out_shape=jax.ShapeDtypeStruct((1, npad), jnp.float32),
        grid_spec=pltpu.PrefetchScalarGridSpec(
            num_scalar_prefetch=0,
            grid=(2, g2),
            in_specs=[
                pl.BlockSpec(memory_space=pl.ANY),
                pl.BlockSpec((ngrp * cin * 128, L), lambda i, j: (0, 0)),
                pl.BlockSpec((P * M, S), lambda i, j: (0, 0)),
                pl.BlockSpec((P * M, S), lambda i, j: (0, 0)),
                pl.BlockSpec((M, 1), lambda i, j: (0, 0)),
                pl.BlockSpec((H, M), lambda i, j: (0, 0)),
                pl.BlockSpec((H, 1), lambda i, j: (0, 0)),
                pl.BlockSpec((1, H), lambda i, j: (0, 0)),
                pl.BlockSpec((1, 1), lambda i, j: (0, 0)),
            ],
            out_specs=pl.BlockSpec((1, block_b),
                                   lambda i, j, g2=g2: (0, i * g2 + j)),
            scratch_shapes=[
                pltpu.VMEM((2, cin, block_b, L), jnp.float32),
                pltpu.SemaphoreType.DMA((2, cin)),
                pltpu.VMEM((R, block_b), jnp.bfloat16),
            ],
        ),
        compiler_params=pltpu.CompilerParams(
            dimension_semantics=("parallel", "arbitrary")),
    )(x, perm, wtoe, wtail, bconv2, whidT, bhid2, wneuT, bneu2)
    return out[0, :N].reshape(N, 1)
